# baseline (speedup 1.0000x reference)
import jax
import jax.numpy as jnp
from jax import lax
from jax.experimental import pallas as pl
from jax.experimental.pallas import tpu as pltpu
from jax.experimental.pallas import tpu_sc as plsc
from jax._src import core as _jax_core
from jax._src.pallas.mosaic import core as _tpu_core
from jax._src.pallas.mosaic import sc_lowering as _sc_lowering
from jax.experimental.mosaic.dialects import tpu as _tpu_dialect

_lane_gather_p = _jax_core.Primitive("learned_pos_emb_lane_gather")
_lane_gather_p.def_abstract_eval(
    lambda x, i: _jax_core.ShapedArray(x.shape, x.dtype))


@_sc_lowering.register_lowering_rule(
    _lane_gather_p, kernel_types=[_tpu_core.CoreType.SC_VECTOR_SUBCORE])
def _lane_gather_rule(ctx, x, idx):
    del ctx
    return _tpu_dialect.dynamic_gather(x, idx, dimensions=[0])


def _lane_splat(x, lane_idx):
    return _lane_gather_p.bind(
        x, jnp.full((_L,), lane_idx, jnp.int32))

_NUM_EMB = 4098
_DIM = 1024
_BATCH = 4
_SEQ = 4096

_NC = 2
_NS = 16
_L = 16
_NW = _NC * _NS

_TOKENS = _BATCH * _SEQ
_COLS_W = _SEQ // _NW
_CH = 16
_NCH = _COLS_W // _CH
_Q = 64


def _body(ids_hbm, table_hbm, out_hbm, ids_v, emb0_v, g0, g1, ob0, ob1,
          gsem0, gsem1, ssem0, ssem1):
    wid = lax.axis_index("s") * _NC + lax.axis_index("c")
    col0 = wid * _COLS_W

    pltpu.sync_copy(table_hbm.at[pl.ds(0, _DIM)], emb0_v)
    for b in range(_BATCH):
        pltpu.sync_copy(ids_hbm.at[pl.ds(b * _SEQ + col0, _COLS_W)],
                        ids_v.at[b])

    gbufs = (g0, g1)
    gsems = (gsem0, gsem1)
    obufs = (ob0, ob1)
    ssems = (ssem0, ssem1)

    def gissue(c_val, gp):
        pltpu.async_copy(
            table_hbm.at[pl.ds((col0 + c_val * _CH + 1) * _DIM, _CH * _DIM)],
            gbufs[gp], gsems[gp])

    def gwait(gp):
        pltpu.make_async_copy(table_hbm.at[pl.ds(0, _CH * _DIM)], gbufs[gp],
                              gsems[gp]).wait()

    def swait(sp):
        pltpu.make_async_copy(obufs[sp], out_hbm.at[pl.ds(0, _CH * _DIM)],
                              ssems[sp]).wait()

    def fix_rows(gp, sp, b, c_val):
        gb = gbufs[gp]
        ob = obufs[sp]
        idchunk = ids_v[b, pl.ds(c_val * _CH, _CH)]
        zf = jnp.where(idchunk == 0, 1.0, 0.0).astype(jnp.float32)
        flags = [_lane_splat(zf, r) for r in range(_CH)]

        def blk(q, carry):
            base = q * _Q
            for j in range(_Q // _L):
                jcol = base + j * _L
                e0v = emb0_v[pl.ds(jcol, _L)]
                for r in range(_CH):
                    gv = gb[pl.ds(r * _DIM + jcol, _L)]
                    ob[pl.ds(r * _DIM + jcol, _L)] = (
                        gv + flags[r] * (e0v - gv))
            return carry

        lax.fori_loop(0, _DIM // _Q, blk, 0)

    gissue(0, 0)
    gissue(1, 1)

    def iter_body(k, carry):
        c0 = k * 2
        for cc in range(2):
            c = c0 + cc
            gwait(cc)
            for b in range(_BATCH):
                u = cc * _BATCH + b
                sp = u % 2
                if u >= 2:
                    swait(sp)
                else:
                    @pl.when(k > 0)
                    def _(sp=sp):
                        swait(sp)
                fix_rows(cc, sp, b, c)
                tok0 = b * _SEQ + col0 + c * _CH
                pltpu.async_copy(obufs[sp],
                                 out_hbm.at[pl.ds(tok0 * _DIM, _CH * _DIM)],
                                 ssems[sp])

            @pl.when(c + 2 < _NCH)
            def _(cc=cc, c=c):
                gissue(c + 2, cc)
        return carry

    lax.fori_loop(0, _NCH // 2, iter_body, 0)
    swait(0)
    swait(1)


@jax.jit
def _lookup(ids_flat, table):
    mesh = plsc.VectorSubcoreMesh(
        core_axis_name="c", subcore_axis_name="s",
        num_cores=_NC, num_subcores=_NS)
    fn = pl.kernel(
        _body,
        out_type=jax.ShapeDtypeStruct((_TOKENS * _DIM,), jnp.float32),
        mesh=mesh,
        scratch_types=[
            pltpu.VMEM((_BATCH, _COLS_W), jnp.int32),
            pltpu.VMEM((_DIM,), jnp.float32),
            pltpu.VMEM((_CH * _DIM,), jnp.float32),
            pltpu.VMEM((_CH * _DIM,), jnp.float32),
            pltpu.VMEM((_CH * _DIM,), jnp.float32),
            pltpu.VMEM((_CH * _DIM,), jnp.float32),
            pltpu.SemaphoreType.DMA,
            pltpu.SemaphoreType.DMA,
            pltpu.SemaphoreType.DMA,
            pltpu.SemaphoreType.DMA,
        ],
    )
    return fn(ids_flat, table)


def kernel(input, embeddings):
    ids_flat = input.astype(jnp.int32).reshape(_TOKENS)
    out = _lookup(ids_flat, embeddings.reshape(_NUM_EMB * _DIM))
    return out.reshape(_BATCH, _SEQ, _DIM)
